# baseline (speedup 1.0000x reference)
import functools
import jax
import jax.numpy as jnp
from jax import lax
from jax.experimental import pallas as pl
from jax.experimental.pallas import tpu as pltpu
from jax.experimental.pallas import tpu_sc as plsc

NUM_VERTS = 2562
SPS = 500
SPS_PAD = 512
B, S, P = 4, 8, 8
N = 4096
M_PAD = S * SPS_PAD
K = 3
BN = 512
NB = N // BN
PAD_COORD = 3.0e4
BIGF = 3.0e38


def _prep_kernel(a0, a1, a2, t0, t1, t2, w, scl, offs24, sph24,
                 opx, opy, opz, opn, orep):
    a, b, c = a0[...], a1[...], a2[...]
    ca, sa = jnp.cos(a), jnp.sin(a)
    cb, sb = jnp.cos(b), jnp.sin(b)
    cc, sc_ = jnp.cos(c), jnp.sin(c)
    R00 = cb * cc
    R01 = -cb * sc_
    R02 = sb
    R10 = sa * sb * cc + ca * sc_
    R11 = -sa * sb * sc_ + ca * cc
    R12 = -sa * cb
    R20 = -ca * sb * cc + sa * sc_
    R21 = ca * sb * sc_ + sa * cc
    R22 = ca * cb
    wv = w[...]
    ws = wv * scl[...]
    cx = jnp.sum(wv * t0[...], axis=1, keepdims=True)
    cy = jnp.sum(wv * t1[...], axis=1, keepdims=True)
    cz = jnp.sum(wv * t2[...], axis=1, keepdims=True)
    A24x = jnp.concatenate([ws * R00, ws * R01, ws * R02], axis=1)
    A24y = jnp.concatenate([ws * R10, ws * R11, ws * R12], axis=1)
    A24z = jnp.concatenate([ws * R20, ws * R21, ws * R22], axis=1)
    d24 = offs24[...] + sph24[...]
    dbar = jnp.mean(d24, axis=1, keepdims=True)
    d24p = d24[:, :SPS_PAD]
    px = jnp.dot(A24x, d24p, preferred_element_type=jnp.float32) + cx
    py = jnp.dot(A24y, d24p, preferred_element_type=jnp.float32) + cy
    pz = jnp.dot(A24z, d24p, preferred_element_type=jnp.float32) + cz
    lane = lax.broadcasted_iota(jnp.int32, (32, SPS_PAD), 1)
    pad = lane >= SPS
    px = jnp.where(pad, PAD_COORD, px)
    py = jnp.where(pad, PAD_COORD, py)
    pz = jnp.where(pad, PAD_COORD, pz)
    opx[...] = px
    opy[...] = py
    opz[...] = pz
    opn[...] = px * px + py * py + pz * pz
    cenx = jnp.dot(A24x, dbar, preferred_element_type=jnp.float32) + cx
    ceny = jnp.dot(A24y, dbar, preferred_element_type=jnp.float32) + cy
    cenz = jnp.dot(A24z, dbar, preferred_element_type=jnp.float32) + cz
    r8 = lax.broadcasted_iota(jnp.int32, (S, S), 0)
    c8 = lax.broadcasted_iota(jnp.int32, (S, S), 1)
    offdiag = (r8 != c8).astype(jnp.float32)
    ones81 = jnp.ones((S, 1), jnp.float32)
    rep_total = jnp.float32(0.0)
    dn = (((1,), (1,)), ((), ()))
    for bi in range(B):
        C = jnp.concatenate(
            [cenx[bi * S:(bi + 1) * S, :],
             ceny[bi * S:(bi + 1) * S, :],
             cenz[bi * S:(bi + 1) * S, :]], axis=1)
        G = lax.dot_general(C, C, dn, preferred_element_type=jnp.float32)
        cn = jnp.sum(C * C, axis=1, keepdims=True)
        cnT = lax.dot_general(ones81, cn, dn,
                              preferred_element_type=jnp.float32)
        d2c = jnp.maximum(cn + cnT - 2.0 * G, 0.0)
        dist = jnp.sqrt(d2c + 1e-12)
        rep = jnp.exp(5.0 * jnp.maximum(0.5 - dist, 0.0)) * offdiag
        rep_total = rep_total + jnp.sum(rep) / jnp.float32(S * (S - 1))
    rep_mean = rep_total / jnp.float32(B)
    lane128 = lax.broadcasted_iota(jnp.int32, (1, 128), 1)
    orep[...] = jnp.where(lane128 == 0, rep_mean, 0.0)


R_TC = 3072
NB_TC = R_TC // BN


def _chamfer_kernel(tref, pxref, pyref, pzref, pnref, cm_out, out, acc):
    nb = pl.program_id(1)
    t = tref[0]
    tx = t[:, 0:1]
    ty = t[:, 1:2]
    tz = t[:, 2:3]
    tn = tx * tx + ty * ty + tz * tz
    px = pxref[0]
    py = pyref[0]
    pz = pzref[0]
    pn = pnref[0]
    cross = tx * px + ty * py + tz * pz
    d2 = jnp.maximum(tn + pn - 2.0 * cross, 0.0)

    bmin = jnp.min(d2, axis=0, keepdims=True)

    @pl.when(nb == 0)
    def _():
        cm_out[0] = bmin
        acc[0] = 0.0

    @pl.when(nb != 0)
    def _():
        cm_out[0] = jnp.minimum(cm_out[0], bmin)

    CH = 128
    bnr = d2.shape[0]
    t1 = jnp.full((bnr, CH), BIGF, jnp.float32)
    t2 = t1
    t3v = t1
    for ci in range(M_PAD // CH):
        seg = d2[:, ci * CH:(ci + 1) * CH]
        e2 = jnp.maximum(t1, seg)
        t1 = jnp.minimum(t1, seg)
        e3 = jnp.maximum(t2, e2)
        t2 = jnp.minimum(t2, e2)
        t3v = jnp.minimum(t3v, e3)
    cand = jnp.concatenate([t1, t2, t3v], axis=1)
    m1 = jnp.min(cand, axis=1, keepdims=True)
    eq1 = cand <= m1
    c1 = jnp.sum(eq1.astype(jnp.float32), axis=1, keepdims=True)
    d2b = jnp.where(eq1, BIGF, cand)
    m2 = jnp.min(d2b, axis=1, keepdims=True)
    eq2 = d2b <= m2
    c2 = jnp.sum(eq2.astype(jnp.float32), axis=1, keepdims=True)
    m3 = jnp.min(jnp.where(eq2, BIGF, d2b), axis=1, keepdims=True)
    k1 = jnp.minimum(c1, 3.0)
    k2 = jnp.minimum(c2, 3.0 - k1)
    k3 = 3.0 - k1 - k2
    t3 = m1 * k1 + m2 * k2 + m3 * k3
    acc[0] = acc[0] + jnp.sum(t3)

    @pl.when(nb == NB_TC - 1)
    def _():
        lane128 = lax.broadcasted_iota(jnp.int32, (1, 8, 128), 2)
        sub8 = lax.broadcasted_iota(jnp.int32, (1, 8, 128), 1)
        out[...] = jnp.where((lane128 == 0) & (sub8 == 0), acc[0], 0.0)


NC, NSUB, L = 2, 16, 16
NTILES = NC * NSUB
R_SC = N - R_TC
TILES_PER_BATCH = NTILES // B
ROWS_PER_TILE = R_SC // TILES_PER_BATCH
RB = 4
NCHUNK = M_PAD // L


def _sc_top3_insert(m1, m2, m3, v):
    e2 = jnp.maximum(m1, v)
    m1 = jnp.minimum(m1, v)
    e3 = jnp.maximum(m2, e2)
    m2 = jnp.minimum(m2, e2)
    m3 = jnp.minimum(m3, e3)
    return m1, m2, m3


def _sc_chamfer(tx_ref, ty_ref, tz_ref, px_ref, py_ref, pz_ref,
                colmin_out, gpart_out,
                txv, tyv, tzv, pxv, pyv, pzv, pnv, cmv,
                b1, b2, b3, stage):
    c = lax.axis_index("c")
    s = lax.axis_index("s")
    wid = c * NSUB + s
    bi = wid // TILES_PER_BATCH
    base = bi * N + R_TC + (wid % TILES_PER_BATCH) * ROWS_PER_TILE
    pltpu.sync_copy(tx_ref.at[pl.ds(base, ROWS_PER_TILE)], txv)
    pltpu.sync_copy(ty_ref.at[pl.ds(base, ROWS_PER_TILE)], tyv)
    pltpu.sync_copy(tz_ref.at[pl.ds(base, ROWS_PER_TILE)], tzv)
    pltpu.sync_copy(px_ref.at[bi], pxv)
    pltpu.sync_copy(py_ref.at[bi], pyv)
    pltpu.sync_copy(pz_ref.at[bi], pzv)

    big = jnp.full((L,), BIGF, jnp.float32)

    def init_cm(j, carry):
        o = j * L
        cmv[pl.ds(o, L)] = big
        x = pxv[pl.ds(o, L)]
        y = pyv[pl.ds(o, L)]
        z = pzv[pl.ds(o, L)]
        pnv[pl.ds(o, L)] = x * x + y * y + z * z
        pxv[pl.ds(o, L)] = -2.0 * x
        pyv[pl.ds(o, L)] = -2.0 * y
        pzv[pl.ds(o, L)] = -2.0 * z
        return carry
    lax.fori_loop(0, NCHUNK, init_cm, 0)

    def row_block(blk, carry0):
        r0 = blk * L
        tx16 = txv[pl.ds(r0, L)]
        ty16 = tyv[pl.ds(r0, L)]
        tz16 = tzv[pl.ds(r0, L)]
        for sub in range(L // RB):
            txs = [tx16[sub * RB + i] for i in range(RB)]
            tys = [ty16[sub * RB + i] for i in range(RB)]
            tzs = [tz16[sub * RB + i] for i in range(RB)]
            tns = [txs[i] * txs[i] + tys[i] * tys[i] + tzs[i] * tzs[i]
                   for i in range(RB)]

            def chunk(j, carry):
                o = j * L
                pxc = pxv[pl.ds(o, L)]
                pyc = pyv[pl.ds(o, L)]
                pzc = pzv[pl.ds(o, L)]
                pnc = pnv[pl.ds(o, L)]
                cm = cmv[pl.ds(o, L)]
                out = []
                for k in range(RB):
                    d2 = pnc + tns[k] + pxc * txs[k] + pyc * tys[k] + pzc * tzs[k]
                    t1, t2, t3 = _sc_top3_insert(
                        carry[3 * k], carry[3 * k + 1], carry[3 * k + 2], d2)
                    out += [t1, t2, t3]
                    cm = jnp.minimum(cm, d2)
                cmv[pl.ds(o, L)] = cm
                return tuple(out)

            tri = lax.fori_loop(0, NCHUNK, chunk,
                                tuple(big for _ in range(3 * RB)))
            for k in range(RB):
                r = r0 + sub * RB + k
                b1[pl.ds(r * L, L)] = tri[3 * k]
                b2[pl.ds(r * L, L)] = tri[3 * k + 1]
                b3[pl.ds(r * L, L)] = tri[3 * k + 2]
        return carry0

    lax.fori_loop(0, ROWS_PER_TILE // L, row_block, 0)

    def merge_row(r, g_acc):
        v1 = b1[pl.ds(r * L, L)]
        v2 = b2[pl.ds(r * L, L)]
        v3 = b3[pl.ds(r * L, L)]
        accs = []
        for v in (v1, v2, v3):
            a1 = jnp.float32(BIGF)
            a2 = jnp.float32(BIGF)
            a3 = jnp.float32(BIGF)
            for i in range(L):
                a1, a2, a3 = _sc_top3_insert(a1, a2, a3, v[i])
            accs.append((a1, a2, a3))
        m1, m2, m3 = accs[0]
        for a in accs[1:]:
            for x in a:
                m1, m2, m3 = _sc_top3_insert(m1, m2, m3, x)
        return g_acc + (m1 + m2 + m3)

    g_total = lax.fori_loop(0, ROWS_PER_TILE, merge_row, jnp.float32(0.0))
    pltpu.sync_copy(cmv, colmin_out.at[wid])
    lane = lax.broadcasted_iota(jnp.int32, (L,), 0)
    stage[...] = jnp.where(lane == 0, g_total, 0.0)
    pltpu.sync_copy(stage, gpart_out.at[wid])


def _finish_kernel(tc_cm_ref, sc_cm_ref, tc_g_ref, sc_g_ref, rep_ref, out):
    cm_sc = jnp.min(sc_cm_ref[...], axis=1)
    cm = jnp.minimum(tc_cm_ref[...], cm_sc)
    lane = lax.broadcasted_iota(jnp.int32, (B, M_PAD), 1)
    valid = (lane % SPS_PAD) < SPS
    s_total = jnp.sum(jnp.where(valid, cm, 0.0))
    g_total = jnp.sum(tc_g_ref[...]) + jnp.sum(sc_g_ref[...])
    rep_loss = rep_ref[0, 0]
    global_loss = g_total / jnp.float32(B * N * K)
    per_slot_loss = s_total / jnp.float32(SPS) / jnp.float32(B * S)
    total = 0.7 * global_loss + 0.3 * per_slot_loss + 0.2 * rep_loss
    out[...] = jnp.full((1, 1), 0.0) + total


def kernel(scales, transforms, prototype_weights, prototype_offsets,
           target_pcls, sphere_verts):
    f32 = jnp.float32
    ang = transforms[..., 3:].reshape(B * S * P, 3)
    trn = transforms[..., :3].reshape(B * S * P, 3)
    a0 = ang[:, 0].reshape(32, 8)
    a1 = ang[:, 1].reshape(32, 8)
    a2 = ang[:, 2].reshape(32, 8)
    t0 = trn[:, 0].reshape(32, 8)
    t1 = trn[:, 1].reshape(32, 8)
    t2 = trn[:, 2].reshape(32, 8)
    w = prototype_weights.reshape(32, 8)
    scl = jnp.broadcast_to(scales.reshape(B, S, 1, 1), (B, S, P, 1)).reshape(32, 8)
    offs24 = prototype_offsets.transpose(2, 0, 1).reshape(24, NUM_VERTS)
    sph24 = jnp.broadcast_to(sphere_verts.T[:, None, :],
                             (3, P, NUM_VERTS)).reshape(24, NUM_VERTS)

    px, py, pz, pn, rep = pl.pallas_call(
        _prep_kernel,
        out_shape=[
            jax.ShapeDtypeStruct((32, SPS_PAD), f32),
            jax.ShapeDtypeStruct((32, SPS_PAD), f32),
            jax.ShapeDtypeStruct((32, SPS_PAD), f32),
            jax.ShapeDtypeStruct((32, SPS_PAD), f32),
            jax.ShapeDtypeStruct((1, 128), f32),
        ],
    )(a0, a1, a2, t0, t1, t2, w, scl, offs24, sph24)

    px = px.reshape(B, M_PAD)
    py = py.reshape(B, M_PAD)
    pz = pz.reshape(B, M_PAD)
    pn = pn.reshape(B, M_PAD)
    tflat = target_pcls.reshape(B * N, 3)
    tx_all = tflat[:, 0]
    ty_all = tflat[:, 1]
    tz_all = tflat[:, 2]

    sc_call = pl.kernel(
        _sc_chamfer,
        out_type=[
            jax.ShapeDtypeStruct((NTILES, M_PAD), f32),
            jax.ShapeDtypeStruct((NTILES, L), f32),
        ],
        mesh=plsc.VectorSubcoreMesh(core_axis_name="c", subcore_axis_name="s",
                                    num_cores=NC, num_subcores=NSUB),
        scratch_types=[
            pltpu.VMEM((ROWS_PER_TILE,), f32),
            pltpu.VMEM((ROWS_PER_TILE,), f32),
            pltpu.VMEM((ROWS_PER_TILE,), f32),
            pltpu.VMEM((M_PAD,), f32),
            pltpu.VMEM((M_PAD,), f32),
            pltpu.VMEM((M_PAD,), f32),
            pltpu.VMEM((M_PAD,), f32),
            pltpu.VMEM((M_PAD,), f32),
            pltpu.VMEM((ROWS_PER_TILE * L,), f32),
            pltpu.VMEM((ROWS_PER_TILE * L,), f32),
            pltpu.VMEM((ROWS_PER_TILE * L,), f32),
            pltpu.VMEM((L,), f32),
        ],
    )
    colmin_sc, gpart_sc = sc_call(tx_all, ty_all, tz_all, px, py, pz)

    plane_spec = pl.BlockSpec((1, 1, M_PAD), lambda b, nb: (b, 0, 0))
    tc_cm, tc_g = pl.pallas_call(
        _chamfer_kernel,
        grid=(B, NB_TC),
        in_specs=[
            pl.BlockSpec((1, BN, 3), lambda b, nb: (b, nb, 0)),
            plane_spec, plane_spec, plane_spec, plane_spec,
        ],
        out_specs=[
            pl.BlockSpec((1, 1, M_PAD), lambda b, nb: (b, 0, 0)),
            pl.BlockSpec((1, 8, 128), lambda b, nb: (b, 0, 0)),
        ],
        out_shape=[
            jax.ShapeDtypeStruct((B, 1, M_PAD), f32),
            jax.ShapeDtypeStruct((B, 8, 128), f32),
        ],
        scratch_shapes=[
            pltpu.SMEM((1,), f32),
        ],
    )(target_pcls, px.reshape(B, 1, M_PAD), py.reshape(B, 1, M_PAD),
      pz.reshape(B, 1, M_PAD), pn.reshape(B, 1, M_PAD))

    out = pl.pallas_call(
        _finish_kernel,
        out_shape=jax.ShapeDtypeStruct((1, 1), f32),
    )(tc_cm.reshape(B, M_PAD), colmin_sc.reshape(B, TILES_PER_BATCH, M_PAD),
      tc_g, gpart_sc, rep)
    return out[0, 0]


# speedup vs baseline: 1.0947x; 1.0947x over previous
import functools
import jax
import jax.numpy as jnp
from jax import lax
from jax.experimental import pallas as pl
from jax.experimental.pallas import tpu as pltpu
from jax.experimental.pallas import tpu_sc as plsc

NUM_VERTS = 2562
SPS = 500
SPS_PAD = 512
B, S, P = 4, 8, 8
N = 4096
M_PAD = S * SPS_PAD
K = 3
BN = 384
NB = N // BN
PAD_COORD = 3.0e4
BIGF = 3.0e38


def _prep_kernel(a0, a1, a2, t0, t1, t2, w, scl, offs24, sph24,
                 opx, opy, opz, opn, orep):
    a, b, c = a0[...], a1[...], a2[...]
    ca, sa = jnp.cos(a), jnp.sin(a)
    cb, sb = jnp.cos(b), jnp.sin(b)
    cc, sc_ = jnp.cos(c), jnp.sin(c)
    R00 = cb * cc
    R01 = -cb * sc_
    R02 = sb
    R10 = sa * sb * cc + ca * sc_
    R11 = -sa * sb * sc_ + ca * cc
    R12 = -sa * cb
    R20 = -ca * sb * cc + sa * sc_
    R21 = ca * sb * sc_ + sa * cc
    R22 = ca * cb
    wv = w[...]
    ws = wv * scl[...]
    cx = jnp.sum(wv * t0[...], axis=1, keepdims=True)
    cy = jnp.sum(wv * t1[...], axis=1, keepdims=True)
    cz = jnp.sum(wv * t2[...], axis=1, keepdims=True)
    A24x = jnp.concatenate([ws * R00, ws * R01, ws * R02], axis=1)
    A24y = jnp.concatenate([ws * R10, ws * R11, ws * R12], axis=1)
    A24z = jnp.concatenate([ws * R20, ws * R21, ws * R22], axis=1)
    d24 = offs24[...] + sph24[...]
    dbar = jnp.mean(d24, axis=1, keepdims=True)
    d24p = d24[:, :SPS_PAD]
    px = jnp.dot(A24x, d24p, preferred_element_type=jnp.float32) + cx
    py = jnp.dot(A24y, d24p, preferred_element_type=jnp.float32) + cy
    pz = jnp.dot(A24z, d24p, preferred_element_type=jnp.float32) + cz
    lane = lax.broadcasted_iota(jnp.int32, (32, SPS_PAD), 1)
    pad = lane >= SPS
    px = jnp.where(pad, PAD_COORD, px)
    py = jnp.where(pad, PAD_COORD, py)
    pz = jnp.where(pad, PAD_COORD, pz)
    opx[...] = px
    opy[...] = py
    opz[...] = pz
    opn[...] = px * px + py * py + pz * pz
    cenx = jnp.dot(A24x, dbar, preferred_element_type=jnp.float32) + cx
    ceny = jnp.dot(A24y, dbar, preferred_element_type=jnp.float32) + cy
    cenz = jnp.dot(A24z, dbar, preferred_element_type=jnp.float32) + cz
    r8 = lax.broadcasted_iota(jnp.int32, (S, S), 0)
    c8 = lax.broadcasted_iota(jnp.int32, (S, S), 1)
    offdiag = (r8 != c8).astype(jnp.float32)
    ones81 = jnp.ones((S, 1), jnp.float32)
    rep_total = jnp.float32(0.0)
    dn = (((1,), (1,)), ((), ()))
    for bi in range(B):
        C = jnp.concatenate(
            [cenx[bi * S:(bi + 1) * S, :],
             ceny[bi * S:(bi + 1) * S, :],
             cenz[bi * S:(bi + 1) * S, :]], axis=1)
        G = lax.dot_general(C, C, dn, preferred_element_type=jnp.float32)
        cn = jnp.sum(C * C, axis=1, keepdims=True)
        cnT = lax.dot_general(ones81, cn, dn,
                              preferred_element_type=jnp.float32)
        d2c = jnp.maximum(cn + cnT - 2.0 * G, 0.0)
        dist = jnp.sqrt(d2c + 1e-12)
        rep = jnp.exp(5.0 * jnp.maximum(0.5 - dist, 0.0)) * offdiag
        rep_total = rep_total + jnp.sum(rep) / jnp.float32(S * (S - 1))
    rep_mean = rep_total / jnp.float32(B)
    lane128 = lax.broadcasted_iota(jnp.int32, (1, 128), 1)
    orep[...] = jnp.where(lane128 == 0, rep_mean, 0.0)


R_TC = 2688
NB_TC = R_TC // BN


def _chamfer_kernel(tref, pxref, pyref, pzref, pnref, cm_out, out, acc):
    nb = pl.program_id(1)
    t = tref[0]
    tx = t[:, 0:1]
    ty = t[:, 1:2]
    tz = t[:, 2:3]
    tn = tx * tx + ty * ty + tz * tz
    px = pxref[0]
    py = pyref[0]
    pz = pzref[0]
    pn = pnref[0]
    cross = tx * px + ty * py + tz * pz
    d2 = jnp.maximum(tn + pn - 2.0 * cross, 0.0)

    bmin = jnp.min(d2, axis=0, keepdims=True)

    @pl.when(nb == 0)
    def _():
        cm_out[0] = bmin
        acc[0] = 0.0

    @pl.when(nb != 0)
    def _():
        cm_out[0] = jnp.minimum(cm_out[0], bmin)

    CH = 128
    bnr = d2.shape[0]
    t1 = jnp.full((bnr, CH), BIGF, jnp.float32)
    t2 = t1
    t3v = t1
    for ci in range(M_PAD // CH):
        seg = d2[:, ci * CH:(ci + 1) * CH]
        e2 = jnp.maximum(t1, seg)
        t1 = jnp.minimum(t1, seg)
        e3 = jnp.maximum(t2, e2)
        t2 = jnp.minimum(t2, e2)
        t3v = jnp.minimum(t3v, e3)
    cand = jnp.concatenate([t1, t2, t3v], axis=1)
    m1 = jnp.min(cand, axis=1, keepdims=True)
    eq1 = cand <= m1
    c1 = jnp.sum(eq1.astype(jnp.float32), axis=1, keepdims=True)
    d2b = jnp.where(eq1, BIGF, cand)
    m2 = jnp.min(d2b, axis=1, keepdims=True)
    eq2 = d2b <= m2
    c2 = jnp.sum(eq2.astype(jnp.float32), axis=1, keepdims=True)
    m3 = jnp.min(jnp.where(eq2, BIGF, d2b), axis=1, keepdims=True)
    k1 = jnp.minimum(c1, 3.0)
    k2 = jnp.minimum(c2, 3.0 - k1)
    k3 = 3.0 - k1 - k2
    t3 = m1 * k1 + m2 * k2 + m3 * k3
    acc[0] = acc[0] + jnp.sum(t3)

    @pl.when(nb == NB_TC - 1)
    def _():
        lane128 = lax.broadcasted_iota(jnp.int32, (1, 8, 128), 2)
        sub8 = lax.broadcasted_iota(jnp.int32, (1, 8, 128), 1)
        out[...] = jnp.where((lane128 == 0) & (sub8 == 0), acc[0], 0.0)


NC, NSUB, L = 2, 16, 16
NTILES = NC * NSUB
R_SC = N - R_TC
TILES_PER_BATCH = NTILES // B
ROWS_PER_TILE = R_SC // TILES_PER_BATCH
RB = 4
NCHUNK = M_PAD // L


def _sc_top3_insert(m1, m2, m3, v):
    e2 = jnp.maximum(m1, v)
    m1 = jnp.minimum(m1, v)
    e3 = jnp.maximum(m2, e2)
    m2 = jnp.minimum(m2, e2)
    m3 = jnp.minimum(m3, e3)
    return m1, m2, m3


def _sc_chamfer(tx_ref, ty_ref, tz_ref, px_ref, py_ref, pz_ref,
                colmin_out, gpart_out,
                txv, tyv, tzv, pxv, pyv, pzv, pnv, cmv,
                b1, b2, b3, stage):
    c = lax.axis_index("c")
    s = lax.axis_index("s")
    wid = c * NSUB + s
    bi = wid // TILES_PER_BATCH
    base = bi * N + R_TC + (wid % TILES_PER_BATCH) * ROWS_PER_TILE
    pltpu.sync_copy(tx_ref.at[pl.ds(base, ROWS_PER_TILE)], txv)
    pltpu.sync_copy(ty_ref.at[pl.ds(base, ROWS_PER_TILE)], tyv)
    pltpu.sync_copy(tz_ref.at[pl.ds(base, ROWS_PER_TILE)], tzv)
    pltpu.sync_copy(px_ref.at[bi], pxv)
    pltpu.sync_copy(py_ref.at[bi], pyv)
    pltpu.sync_copy(pz_ref.at[bi], pzv)

    big = jnp.full((L,), BIGF, jnp.float32)

    def init_cm(j, carry):
        o = j * L
        cmv[pl.ds(o, L)] = big
        x = pxv[pl.ds(o, L)]
        y = pyv[pl.ds(o, L)]
        z = pzv[pl.ds(o, L)]
        pnv[pl.ds(o, L)] = x * x + y * y + z * z
        pxv[pl.ds(o, L)] = -2.0 * x
        pyv[pl.ds(o, L)] = -2.0 * y
        pzv[pl.ds(o, L)] = -2.0 * z
        return carry
    lax.fori_loop(0, NCHUNK, init_cm, 0)

    def row_block(blk, carry0):
        r0 = blk * L
        tx16 = txv[pl.ds(r0, L)]
        ty16 = tyv[pl.ds(r0, L)]
        tz16 = tzv[pl.ds(r0, L)]
        for sub in range(L // RB):
            txs = [tx16[sub * RB + i] for i in range(RB)]
            tys = [ty16[sub * RB + i] for i in range(RB)]
            tzs = [tz16[sub * RB + i] for i in range(RB)]
            tns = [txs[i] * txs[i] + tys[i] * tys[i] + tzs[i] * tzs[i]
                   for i in range(RB)]

            def chunk(j, carry):
                o = j * L
                pxc = pxv[pl.ds(o, L)]
                pyc = pyv[pl.ds(o, L)]
                pzc = pzv[pl.ds(o, L)]
                pnc = pnv[pl.ds(o, L)]
                cm = cmv[pl.ds(o, L)]
                out = []
                for k in range(RB):
                    d2 = pnc + tns[k] + pxc * txs[k] + pyc * tys[k] + pzc * tzs[k]
                    t1, t2, t3 = _sc_top3_insert(
                        carry[3 * k], carry[3 * k + 1], carry[3 * k + 2], d2)
                    out += [t1, t2, t3]
                    cm = jnp.minimum(cm, d2)
                cmv[pl.ds(o, L)] = cm
                return tuple(out)

            tri = lax.fori_loop(0, NCHUNK, chunk,
                                tuple(big for _ in range(3 * RB)))
            for k in range(RB):
                r = r0 + sub * RB + k
                b1[pl.ds(r * L, L)] = tri[3 * k]
                b2[pl.ds(r * L, L)] = tri[3 * k + 1]
                b3[pl.ds(r * L, L)] = tri[3 * k + 2]
        return carry0

    lax.fori_loop(0, ROWS_PER_TILE // L, row_block, 0)

    def merge_row(r, g_acc):
        v1 = b1[pl.ds(r * L, L)]
        v2 = b2[pl.ds(r * L, L)]
        v3 = b3[pl.ds(r * L, L)]
        accs = []
        for v in (v1, v2, v3):
            a1 = jnp.float32(BIGF)
            a2 = jnp.float32(BIGF)
            a3 = jnp.float32(BIGF)
            for i in range(L):
                a1, a2, a3 = _sc_top3_insert(a1, a2, a3, v[i])
            accs.append((a1, a2, a3))
        m1, m2, m3 = accs[0]
        for a in accs[1:]:
            for x in a:
                m1, m2, m3 = _sc_top3_insert(m1, m2, m3, x)
        return g_acc + (m1 + m2 + m3)

    g_total = lax.fori_loop(0, ROWS_PER_TILE, merge_row, jnp.float32(0.0))
    pltpu.sync_copy(cmv, colmin_out.at[wid])
    lane = lax.broadcasted_iota(jnp.int32, (L,), 0)
    stage[...] = jnp.where(lane == 0, g_total, 0.0)
    pltpu.sync_copy(stage, gpart_out.at[wid])


def _finish_kernel(tc_cm_ref, sc_cm_ref, tc_g_ref, sc_g_ref, rep_ref, out):
    cm_sc = jnp.min(sc_cm_ref[...], axis=1)
    cm = jnp.minimum(tc_cm_ref[...], cm_sc)
    lane = lax.broadcasted_iota(jnp.int32, (B, M_PAD), 1)
    valid = (lane % SPS_PAD) < SPS
    s_total = jnp.sum(jnp.where(valid, cm, 0.0))
    g_total = jnp.sum(tc_g_ref[...]) + jnp.sum(sc_g_ref[...])
    rep_loss = rep_ref[0, 0]
    global_loss = g_total / jnp.float32(B * N * K)
    per_slot_loss = s_total / jnp.float32(SPS) / jnp.float32(B * S)
    total = 0.7 * global_loss + 0.3 * per_slot_loss + 0.2 * rep_loss
    out[...] = jnp.full((1, 1), 0.0) + total


def kernel(scales, transforms, prototype_weights, prototype_offsets,
           target_pcls, sphere_verts):
    f32 = jnp.float32
    ang = transforms[..., 3:].reshape(B * S * P, 3)
    trn = transforms[..., :3].reshape(B * S * P, 3)
    a0 = ang[:, 0].reshape(32, 8)
    a1 = ang[:, 1].reshape(32, 8)
    a2 = ang[:, 2].reshape(32, 8)
    t0 = trn[:, 0].reshape(32, 8)
    t1 = trn[:, 1].reshape(32, 8)
    t2 = trn[:, 2].reshape(32, 8)
    w = prototype_weights.reshape(32, 8)
    scl = jnp.broadcast_to(scales.reshape(B, S, 1, 1), (B, S, P, 1)).reshape(32, 8)
    offs24 = prototype_offsets.transpose(2, 0, 1).reshape(24, NUM_VERTS)
    sph24 = jnp.broadcast_to(sphere_verts.T[:, None, :],
                             (3, P, NUM_VERTS)).reshape(24, NUM_VERTS)

    px, py, pz, pn, rep = pl.pallas_call(
        _prep_kernel,
        out_shape=[
            jax.ShapeDtypeStruct((32, SPS_PAD), f32),
            jax.ShapeDtypeStruct((32, SPS_PAD), f32),
            jax.ShapeDtypeStruct((32, SPS_PAD), f32),
            jax.ShapeDtypeStruct((32, SPS_PAD), f32),
            jax.ShapeDtypeStruct((1, 128), f32),
        ],
    )(a0, a1, a2, t0, t1, t2, w, scl, offs24, sph24)

    px = px.reshape(B, M_PAD)
    py = py.reshape(B, M_PAD)
    pz = pz.reshape(B, M_PAD)
    pn = pn.reshape(B, M_PAD)
    tflat = target_pcls.reshape(B * N, 3)
    tx_all = tflat[:, 0]
    ty_all = tflat[:, 1]
    tz_all = tflat[:, 2]

    sc_call = pl.kernel(
        _sc_chamfer,
        out_type=[
            jax.ShapeDtypeStruct((NTILES, M_PAD), f32),
            jax.ShapeDtypeStruct((NTILES, L), f32),
        ],
        mesh=plsc.VectorSubcoreMesh(core_axis_name="c", subcore_axis_name="s",
                                    num_cores=NC, num_subcores=NSUB),
        scratch_types=[
            pltpu.VMEM((ROWS_PER_TILE,), f32),
            pltpu.VMEM((ROWS_PER_TILE,), f32),
            pltpu.VMEM((ROWS_PER_TILE,), f32),
            pltpu.VMEM((M_PAD,), f32),
            pltpu.VMEM((M_PAD,), f32),
            pltpu.VMEM((M_PAD,), f32),
            pltpu.VMEM((M_PAD,), f32),
            pltpu.VMEM((M_PAD,), f32),
            pltpu.VMEM((ROWS_PER_TILE * L,), f32),
            pltpu.VMEM((ROWS_PER_TILE * L,), f32),
            pltpu.VMEM((ROWS_PER_TILE * L,), f32),
            pltpu.VMEM((L,), f32),
        ],
    )
    colmin_sc, gpart_sc = sc_call(tx_all, ty_all, tz_all, px, py, pz)

    plane_spec = pl.BlockSpec((1, 1, M_PAD), lambda b, nb: (b, 0, 0))
    tc_cm, tc_g = pl.pallas_call(
        _chamfer_kernel,
        grid=(B, NB_TC),
        in_specs=[
            pl.BlockSpec((1, BN, 3), lambda b, nb: (b, nb, 0)),
            plane_spec, plane_spec, plane_spec, plane_spec,
        ],
        out_specs=[
            pl.BlockSpec((1, 1, M_PAD), lambda b, nb: (b, 0, 0)),
            pl.BlockSpec((1, 8, 128), lambda b, nb: (b, 0, 0)),
        ],
        out_shape=[
            jax.ShapeDtypeStruct((B, 1, M_PAD), f32),
            jax.ShapeDtypeStruct((B, 8, 128), f32),
        ],
        scratch_shapes=[
            pltpu.SMEM((1,), f32),
        ],
    )(target_pcls, px.reshape(B, 1, M_PAD), py.reshape(B, 1, M_PAD),
      pz.reshape(B, 1, M_PAD), pn.reshape(B, 1, M_PAD))

    out = pl.pallas_call(
        _finish_kernel,
        out_shape=jax.ShapeDtypeStruct((1, 1), f32),
    )(tc_cm.reshape(B, M_PAD), colmin_sc.reshape(B, TILES_PER_BATCH, M_PAD),
      tc_g, gpart_sc, rep)
    return out[0, 0]
